# baseline (speedup 1.0000x reference)
import functools
import math

import numpy as np
import jax
import jax.numpy as jnp
from jax import lax
from jax.experimental import pallas as pl
from jax.experimental.pallas import tpu as pltpu
from jax.experimental.pallas import tpu_sc as plsc

_D = 1024
_SCALE = math.sqrt(_D)
_NC, _NS = 2, 16
_NW = _NC * _NS
_K = 16
_LANES = 16


def _make_pe(seq, d):
    position = np.arange(0, seq, dtype=np.float32)[:, None]
    div_term = np.exp(
        np.arange(0, d, 2, dtype=np.float32) * (-math.log(10000.0) / d))
    pe = np.zeros((seq, d), dtype=np.float32)
    pe[:, 0::2] = np.sin(position * div_term)
    pe[:, 1::2] = np.cos(position * div_term)
    return pe


@functools.partial(jax.jit, static_argnames=("seq",))
def _sc_embed(x_flat, table, pe, seq):
    n = x_flat.shape[0]
    npw = n // _NW
    nchunk = npw // _K
    mesh = plsc.VectorSubcoreMesh(core_axis_name="c", subcore_axis_name="s")

    @functools.partial(
        pl.kernel,
        out_type=jax.ShapeDtypeStruct((n, _D), jnp.float32),
        mesh=mesh,
        scratch_types=[
            pltpu.VMEM((npw,), jnp.int32),
            pltpu.VMEM((_K, _D), jnp.float32),
            pltpu.VMEM((_K, _D), jnp.float32),
            pltpu.SemaphoreType.DMA,
        ],
    )
    def k(x_hbm, table_hbm, pe_hbm, out_hbm, idx_v, rows_v, pe_v, sem):
        wid = lax.axis_index("s") * _NC + lax.axis_index("c")
        base = wid * npw
        pos0 = lax.rem(base, seq)
        pltpu.sync_copy(x_hbm.at[pl.ds(base, npw)], idx_v)

        def chunk_body(ci, carry):
            row0 = ci * _K
            pltpu.async_copy(
                table_hbm.at[idx_v.at[pl.ds(row0, _K)]], rows_v, sem).wait()
            pltpu.sync_copy(pe_hbm.at[pl.ds(pos0 + row0, _K)], pe_v)

            def row_body(r, carry2):
                def col_body(j, carry3):
                    sl = pl.ds(j * _LANES, _LANES)
                    rows_v[r, sl] = rows_v[r, sl] * _SCALE + pe_v[r, sl]
                    return carry3
                return lax.fori_loop(0, _D // _LANES, col_body, carry2)

            lax.fori_loop(0, _K, row_body, 0)
            pltpu.sync_copy(rows_v, out_hbm.at[pl.ds(base + row0, _K)])
            return carry

        lax.fori_loop(0, nchunk, chunk_body, 0)

    return k(x_flat, table, pe)


def kernel(x, table):
    b, s = x.shape
    pe = jnp.asarray(_make_pe(s, _D))
    x_flat = x.reshape(-1).astype(jnp.int32)
    out = _sc_embed(x_flat, table, pe, s)
    return out.reshape(b, s, _D)


# speedup vs baseline: 2.4058x; 2.4058x over previous
import functools
import math

import numpy as np
import jax
import jax.numpy as jnp
from jax import lax
from jax.experimental import pallas as pl
from jax.experimental.pallas import tpu as pltpu
from jax.experimental.pallas import tpu_sc as plsc

_D = 1024
_SCALE = math.sqrt(_D)
_NC, _NS = 2, 16
_NW = _NC * _NS
_K = 8
_LANES = 16
_CPR = _D // _LANES
_UNROLL = 16


def _make_pe(seq, d):
    position = np.arange(0, seq, dtype=np.float32)[:, None]
    div_term = np.exp(
        np.arange(0, d, 2, dtype=np.float32) * (-math.log(10000.0) / d))
    pe = np.zeros((seq, d), dtype=np.float32)
    pe[:, 0::2] = np.sin(position * div_term)
    pe[:, 1::2] = np.cos(position * div_term)
    return pe


@functools.partial(jax.jit, static_argnames=("seq",))
def _sc_embed(x_flat, table, pe, seq):
    n = x_flat.shape[0]
    npw = n // _NW
    nchunk = npw // _K
    mesh = plsc.VectorSubcoreMesh(core_axis_name="c", subcore_axis_name="s")

    @functools.partial(
        pl.kernel,
        out_type=jax.ShapeDtypeStruct((n, _D), jnp.float32),
        mesh=mesh,
        scratch_types=[
            pltpu.VMEM((npw,), jnp.int32),
            pltpu.VMEM((_K, _D), jnp.float32),
            pltpu.VMEM((_K, _D), jnp.float32),
            pltpu.VMEM((_K, _D), jnp.float32),
            pltpu.VMEM((_K, _D), jnp.float32),
            pltpu.SemaphoreType.DMA,
            pltpu.SemaphoreType.DMA,
            pltpu.SemaphoreType.DMA,
            pltpu.SemaphoreType.DMA,
            pltpu.SemaphoreType.DMA,
            pltpu.SemaphoreType.DMA,
        ],
    )
    def k(x_hbm, table_hbm, pe_hbm, out_hbm, idx_v, rows0, rows1,
          po0, po1, g0, g1, p0, p1, o0, o1):
        wid = lax.axis_index("s") * _NC + lax.axis_index("c")
        base = wid * npw
        pos0 = lax.rem(base, seq)
        pltpu.sync_copy(x_hbm.at[pl.ds(base, npw)], idx_v)

        rows = [rows0, rows1]
        po = [po0, po1]
        gsem = [g0, g1]
        psem = [p0, p1]
        osem = [o0, o1]

        def start_fetch(ci, b):
            gd = pltpu.async_copy(
                table_hbm.at[idx_v.at[pl.ds(ci * _K, _K)]], rows[b], gsem[b])
            pd = pltpu.async_copy(
                pe_hbm.at[pl.ds(pos0 + ci * _K, _K)], po[b], psem[b])
            return gd, pd

        g = [None, None]
        p = [None, None]
        o = [None, None]
        g[0], p[0] = start_fetch(0, 0)

        for ci in range(nchunk):
            b = ci & 1
            nb = b ^ 1
            if o[nb] is not None:
                o[nb].wait()
            if ci + 1 < nchunk:
                g[nb], p[nb] = start_fetch(ci + 1, nb)
            g[b].wait()
            p[b].wait()

            rows_b = rows[b]
            po_b = po[b]

            def compute_body(it, carry):
                i0 = it * _UNROLL
                for u in range(_UNROLL):
                    i = i0 + u
                    r = lax.shift_right_logical(i, 6)
                    j = lax.bitwise_and(i, _CPR - 1)
                    sl = pl.ds(j * _LANES, _LANES)
                    plsc.addupdate(po_b.at[r, sl], rows_b[r, sl] * _SCALE)
                return carry

            lax.fori_loop(0, _K * _CPR // _UNROLL, compute_body, 0)

            o[b] = pltpu.async_copy(
                po[b], out_hbm.at[pl.ds(base + ci * _K, _K)], osem[b])

        o[(nchunk - 1) & 1].wait()

    return k(x_flat, table, pe)


def kernel(x, table):
    b, s = x.shape
    pe = jnp.asarray(_make_pe(s, _D))
    x_flat = x.reshape(-1).astype(jnp.int32)
    out = _sc_embed(x_flat, table, pe, s)
    return out.reshape(b, s, _D)
